# baseline (speedup 1.0000x reference)
import math

import jax
import jax.numpy as jnp
import numpy as np
from jax import lax
from jax.experimental import pallas as pl
from jax.experimental.pallas import tpu as pltpu
from jax.experimental.pallas import tpu_sc as plsc

NUM_VOCAB = 1000000
EMBED_DIM = 64
MAXLEN = 512
BATCH = 1024
SEQLEN = 200

NC = 2
NS = 16
NW = NC * NS

TOTAL = BATCH * SEQLEN
CHUNK = 128
PER_WORKER = TOTAL // NW
CHUNKS_PER_WORKER = PER_WORKER // CHUNK


def _make_pe(maxlen, d_model):
    position = np.arange(maxlen, dtype=np.float32)[:, None]
    div_term = np.exp(
        np.arange(0, d_model, 2).astype(np.float32) * (-math.log(10000.0) / d_model)
    )
    pe = np.zeros((maxlen, d_model), dtype=np.float32)
    pe[:, 0::2] = np.sin(position * div_term)
    pe[:, 1::2] = np.cos(position * div_term)
    return pe


_PE = _make_pe(MAXLEN, EMBED_DIM)[:SEQLEN]


def _sc_embed(idx_flat, table, pe):
    mesh = plsc.VectorSubcoreMesh(core_axis_name="c", subcore_axis_name="s")

    @pl.kernel(
        out_type=jax.ShapeDtypeStruct((TOTAL, EMBED_DIM), jnp.float32),
        mesh=mesh,
        compiler_params=pltpu.CompilerParams(use_tc_tiling_on_sc=False),
        scratch_types=[
            pltpu.VMEM((PER_WORKER,), jnp.int32),
            pltpu.VMEM((SEQLEN, EMBED_DIM), jnp.float32),
            pltpu.VMEM((CHUNK, EMBED_DIM), jnp.float32),
            pltpu.SemaphoreType.DMA,
        ],
    )
    def k(idx_hbm, table_hbm, pe_hbm, out_hbm, idx_v, pe_v, g, sem):
        wid = lax.axis_index("s") * NC + lax.axis_index("c")
        pltpu.sync_copy(idx_hbm.at[pl.ds(wid * PER_WORKER, PER_WORKER)], idx_v)
        pltpu.sync_copy(pe_hbm, pe_v)

        def chunk_body(j, carry):
            pltpu.async_copy(
                table_hbm.at[idx_v.at[pl.ds(j * CHUNK, CHUNK)]], g, sem
            ).wait()
            pos0 = lax.rem(j * CHUNK, SEQLEN)

            def row_body(r, pos):
                for d in range(EMBED_DIM // 16):
                    sl = pl.ds(d * 16, 16)
                    g[r, sl] = g[r, sl] * 8.0 + pe_v[pos, sl]
                nxt = pos + 1
                return lax.select(nxt == SEQLEN, 0, nxt)

            lax.fori_loop(0, CHUNK, row_body, pos0)
            base = (wid * CHUNKS_PER_WORKER + j) * CHUNK
            pltpu.sync_copy(g, out_hbm.at[pl.ds(base, CHUNK)])
            return carry

        lax.fori_loop(0, CHUNKS_PER_WORKER, chunk_body, 0)

    return k(idx_flat, table, pe)


def kernel(tokens, table):
    idx_flat = tokens.reshape(TOTAL).astype(jnp.int32)
    out = _sc_embed(idx_flat, table, jnp.asarray(_PE))
    return out.reshape(BATCH, SEQLEN, EMBED_DIM)
